# baseline (speedup 1.0000x reference)
import functools

import jax
import jax.numpy as jnp
from jax import lax
from jax.experimental import pallas as pl
from jax.experimental.pallas import tpu as pltpu
from jax.experimental.pallas import tpu_sc as plsc

_EPS = 1e-4
_MAX_LAPLACE_VAR = 1000.0
_MAX_DENSITY = 10000.0

_H = 512
_W = 512
_N = float(_H * _W)
_LANES = 16
_HB = 8
_QROWS = (_H // 4) // 4


def _rgb_kernel(rgb_ref, ids_ref, stats_ref):
    r = rgb_ref[:, 0]
    g = rgb_ref[:, 1]
    b = rgb_ref[:, 2]
    gray = 0.299 * r + 0.587 * g + 0.114 * b

    zrow = jnp.zeros((2, 1, _W), jnp.float32)
    zcol = jnp.zeros((2, _H, 1), jnp.float32)
    dn = jnp.concatenate([gray[:, 1:], zrow], axis=1)
    up = jnp.concatenate([zrow, gray[:, :-1]], axis=1)
    rt = jnp.concatenate([gray[:, :, 1:], zcol], axis=2)
    lt = jnp.concatenate([zcol, gray[:, :, :-1]], axis=2)
    lap = dn + up + rt + lt - 4.0 * gray
    lsum = jnp.sum(jnp.sum(lap, axis=2), axis=1)
    lsq = jnp.sum(jnp.sum(lap * lap, axis=2), axis=1)

    ids = jnp.clip(gray * 255.0, 0.0, 255.0).astype(jnp.int32)
    idr = ids.reshape(2, 4, _H // 4, _W)
    packed = (idr[:, 0] | (idr[:, 1] << 8)
              | (idr[:, 2] << 16) | (idr[:, 3] << 24))
    ids_ref[...] = packed

    lane = lax.broadcasted_iota(jnp.int32, (2, 1, 128), 2)
    vals = (jnp.where(lane == 0, lsum.reshape(2, 1, 1), 0.0)
            + jnp.where(lane == 1, lsq.reshape(2, 1, 1), 0.0))
    stats_ref[...] = vals


def _tc_rgb(rgb_img, off):
    return pl.pallas_call(
        _rgb_kernel,
        grid=(_HB // 2,),
        in_specs=[
            pl.BlockSpec((2, 3, _H, _W), lambda i: (i + off, 0, 0, 0)),
        ],
        out_specs=[
            pl.BlockSpec((2, _H // 4, _W), lambda i: (i, 0, 0)),
            pl.BlockSpec((2, 1, 128), lambda i: (i, 0, 0)),
        ],
        out_shape=[
            jax.ShapeDtypeStruct((_HB, _H // 4, _W), jnp.int32),
            jax.ShapeDtypeStruct((_HB, 1, 128), jnp.float32),
        ],
    )(rgb_img)


def _depth_kernel(depth_ref, stats_ref):
    d = depth_ref[:, 0]
    m = d > 0.0
    n = jnp.sum(jnp.sum(m.astype(jnp.float32), axis=2), axis=1)
    s1 = jnp.sum(jnp.sum(jnp.where(m, d, 0.0), axis=2), axis=1)
    s2 = jnp.sum(jnp.sum(jnp.where(m, d * d, 0.0), axis=2), axis=1)

    lane = lax.broadcasted_iota(jnp.int32, (4, 1, 128), 2)
    vals = (jnp.where(lane == 0, n.reshape(4, 1, 1), 0.0)
            + jnp.where(lane == 1, s1.reshape(4, 1, 1), 0.0)
            + jnp.where(lane == 2, s2.reshape(4, 1, 1), 0.0))
    stats_ref[...] = vals


def _tc_depth(depth_img):
    B = depth_img.shape[0]
    return pl.pallas_call(
        _depth_kernel,
        grid=(B // 4,),
        in_specs=[
            pl.BlockSpec((4, 1, _H, _W), lambda i: (i, 0, 0, 0)),
        ],
        out_specs=[
            pl.BlockSpec((4, 1, 128), lambda i: (i, 0, 0)),
        ],
        out_shape=[
            jax.ShapeDtypeStruct((B, 1, 128), jnp.float32),
        ],
    )(depth_img)


def _sc_hist_kernel(ids_hbm, hist_hbm, ids_v, tab_v):
    wid = lax.axis_index("s") * 2 + lax.axis_index("c")
    batch = wid % _HB
    quarter = wid // _HB

    pltpu.sync_copy(ids_hbm.at[batch, pl.ds(quarter * _QROWS, _QROWS)], ids_v)

    def zero_body(i, c):
        tab_v[pl.ds(i * _LANES, _LANES)] = jnp.zeros((_LANES,), jnp.int32)
        return c
    lax.fori_loop(0, 256 // _LANES, zero_body, 0)

    ones = jnp.ones((_LANES,), jnp.int32)
    mask = jnp.full((_LANES,), 255, jnp.int32)

    def body(j, c):
        r = j // 4
        c0 = (j % 4) * (_W // 4)
        ws = [ids_v[r, pl.ds(c0 + cc * _LANES, _LANES)]
              for cc in range(_W // _LANES // 4)]
        for w in ws:
            for k in range(4):
                bk = lax.shift_right_logical(w, 8 * k) & mask
                plsc.addupdate_scatter(tab_v, [bk], ones)
        return c
    lax.fori_loop(0, _QROWS * 4, body, 0)

    pltpu.sync_copy(tab_v, hist_hbm.at[quarter * _HB + batch])


def _sc_hist(ids_packed):
    kern = functools.partial(
        pl.kernel,
        out_type=jax.ShapeDtypeStruct((4 * _HB, 256), jnp.int32),
        mesh=plsc.VectorSubcoreMesh(core_axis_name="c", subcore_axis_name="s"),
        compiler_params=pltpu.CompilerParams(needs_layout_passes=False),
        scratch_types=[
            pltpu.VMEM((_QROWS, _W), jnp.int32),
            pltpu.VMEM((256,), jnp.int32),
        ],
    )(_sc_hist_kernel)
    return kern(ids_packed)


def _combine_kernel(ha_ref, hb_ref, lsa_ref, lsb_ref, dstats_ref,
                    wr_ref, wd_ref, ha_v, hb_v, sem_a, sem_b):
    ca = pltpu.make_async_copy(ha_ref, ha_v, sem_a)
    cb = pltpu.make_async_copy(hb_ref, hb_v, sem_b)
    ca.start()
    cb.start()
    ca.wait()
    cb.wait()
    ha = ha_v[...].astype(jnp.float32)
    hb = hb_v[...].astype(jnp.float32)
    ha8 = ha[0:8] + ha[8:16] + ha[16:24] + ha[24:32]
    hb8 = hb[0:8] + hb[8:16] + hb[16:24] + hb[24:32]
    h16 = jnp.concatenate([ha8, hb8], axis=0)
    p = h16 * (1.0 / _N)
    ent = -jnp.sum(p * (jnp.log(p + _EPS) * (1.0 / 0.6931471805599453)),
                   axis=1, keepdims=True)

    s = jnp.concatenate([jnp.squeeze(lsa_ref[...], 1),
                         jnp.squeeze(lsb_ref[...], 1)], axis=0)
    t = jnp.squeeze(dstats_ref[...], 1)
    lsum = s[:, 0:1]
    lsq = s[:, 1:2]
    n = t[:, 0:1]
    s1 = t[:, 1:2]
    s2 = t[:, 2:3]

    lapvar = (lsq - lsum * lsum * (1.0 / _N)) / (_N - 1.0)
    clarity = lapvar / (_MAX_LAPLACE_VAR + _EPS)
    uniformity = 1.0 / (ent + _EPS)
    rgb_score = 0.5 * (clarity + uniformity)

    mean = s1 / jnp.maximum(n, 1.0)
    var = (s2 - 2.0 * mean * s1 + mean * mean * n) / jnp.maximum(n - 1.0, 1.0)
    noise = jnp.where(n > 0.0, jnp.sqrt(var), 1.0)
    density = n * (1.0 / _N)
    depth_score = 0.5 * (density / (_MAX_DENSITY + _EPS) + 1.0 / (noise + _EPS))

    den = rgb_score + depth_score + _EPS
    wr_ref[...] = rgb_score / den
    wd_ref[...] = depth_score / den


def _combine(hist_a, hist_b, lstats_a, lstats_b, dstats):
    return pl.pallas_call(
        _combine_kernel,
        in_specs=[
            pl.BlockSpec(memory_space=pl.ANY),
            pl.BlockSpec(memory_space=pl.ANY),
            pl.BlockSpec((_HB, 1, 128), lambda: (0, 0, 0)),
            pl.BlockSpec((_HB, 1, 128), lambda: (0, 0, 0)),
            pl.BlockSpec((2 * _HB, 1, 128), lambda: (0, 0, 0)),
        ],
        scratch_shapes=[
            pltpu.VMEM((4 * _HB, 256), jnp.int32),
            pltpu.VMEM((4 * _HB, 256), jnp.int32),
            pltpu.SemaphoreType.DMA,
            pltpu.SemaphoreType.DMA,
        ],
        out_shape=[
            jax.ShapeDtypeStruct((16, 1), jnp.float32),
            jax.ShapeDtypeStruct((16, 1), jnp.float32),
        ],
    )(hist_a, hist_b, lstats_a, lstats_b, dstats)


def kernel(rgb_img, depth_img):
    ids_a, lstats_a = _tc_rgb(rgb_img, 0)
    hist_a = _sc_hist(ids_a)
    ids_b, lstats_b = _tc_rgb(rgb_img, _HB // 2)
    hist_b = _sc_hist(ids_b)
    dstats, = _tc_depth(depth_img)
    wr, wd = _combine(hist_a, hist_b, lstats_a, lstats_b, dstats)
    return wr, wd


# speedup vs baseline: 1.0149x; 1.0149x over previous
import functools

import jax
import jax.numpy as jnp
from jax import lax
from jax.experimental import pallas as pl
from jax.experimental.pallas import tpu as pltpu
from jax.experimental.pallas import tpu_sc as plsc

_EPS = 1e-4
_MAX_LAPLACE_VAR = 1000.0
_MAX_DENSITY = 10000.0

_H = 512
_W = 512
_N = float(_H * _W)
_LANES = 16
_HB = 8
_QROWS = (_H // 4) // 4


def _rgb_kernel(rgb_ref, ids_ref, stats_ref):
    r = rgb_ref[:, 0]
    g = rgb_ref[:, 1]
    b = rgb_ref[:, 2]
    gray = 0.299 * r + 0.587 * g + 0.114 * b

    zrow = jnp.zeros((2, 1, _W), jnp.float32)
    zcol = jnp.zeros((2, _H, 1), jnp.float32)
    dn = jnp.concatenate([gray[:, 1:], zrow], axis=1)
    up = jnp.concatenate([zrow, gray[:, :-1]], axis=1)
    rt = jnp.concatenate([gray[:, :, 1:], zcol], axis=2)
    lt = jnp.concatenate([zcol, gray[:, :, :-1]], axis=2)
    lap = dn + up + rt + lt - 4.0 * gray
    lsum = jnp.sum(jnp.sum(lap, axis=2), axis=1)
    lsq = jnp.sum(jnp.sum(lap * lap, axis=2), axis=1)

    ids = jnp.clip(gray * 255.0, 0.0, 255.0).astype(jnp.int32)
    idr = ids.reshape(2, 4, _H // 4, _W)
    packed = (idr[:, 0] | (idr[:, 1] << 8)
              | (idr[:, 2] << 16) | (idr[:, 3] << 24))
    ids_ref[...] = packed

    lane = lax.broadcasted_iota(jnp.int32, (2, 1, 128), 2)
    vals = (jnp.where(lane == 0, lsum.reshape(2, 1, 1), 0.0)
            + jnp.where(lane == 1, lsq.reshape(2, 1, 1), 0.0))
    stats_ref[...] = vals


def _tc_rgb(rgb_img, off):
    return pl.pallas_call(
        _rgb_kernel,
        grid=(_HB // 2,),
        in_specs=[
            pl.BlockSpec((2, 3, _H, _W), lambda i: (i + off, 0, 0, 0)),
        ],
        out_specs=[
            pl.BlockSpec((2, _H // 4, _W), lambda i: (i, 0, 0)),
            pl.BlockSpec((2, 1, 128), lambda i: (i, 0, 0)),
        ],
        out_shape=[
            jax.ShapeDtypeStruct((_HB, _H // 4, _W), jnp.int32),
            jax.ShapeDtypeStruct((_HB, 1, 128), jnp.float32),
        ],
    )(rgb_img)


def _depth_kernel(depth_ref, stats_ref):
    d = depth_ref[:, 0]
    m = d > 0.0
    n = jnp.sum(jnp.sum(m.astype(jnp.float32), axis=2), axis=1)
    s1 = jnp.sum(jnp.sum(jnp.where(m, d, 0.0), axis=2), axis=1)
    s2 = jnp.sum(jnp.sum(jnp.where(m, d * d, 0.0), axis=2), axis=1)

    lane = lax.broadcasted_iota(jnp.int32, (4, 1, 128), 2)
    vals = (jnp.where(lane == 0, n.reshape(4, 1, 1), 0.0)
            + jnp.where(lane == 1, s1.reshape(4, 1, 1), 0.0)
            + jnp.where(lane == 2, s2.reshape(4, 1, 1), 0.0))
    stats_ref[...] = vals


def _tc_depth(depth_img):
    B = depth_img.shape[0]
    return pl.pallas_call(
        _depth_kernel,
        grid=(B // 4,),
        in_specs=[
            pl.BlockSpec((4, 1, _H, _W), lambda i: (i, 0, 0, 0)),
        ],
        out_specs=[
            pl.BlockSpec((4, 1, 128), lambda i: (i, 0, 0)),
        ],
        out_shape=[
            jax.ShapeDtypeStruct((B, 1, 128), jnp.float32),
        ],
    )(depth_img)


def _sc_hist_kernel(ids_hbm, hist_hbm, ids_v, tab_v):
    wid = lax.axis_index("s") * 2 + lax.axis_index("c")
    batch = wid % _HB
    quarter = wid // _HB

    pltpu.sync_copy(ids_hbm.at[batch, pl.ds(quarter * _QROWS, _QROWS)], ids_v)

    def zero_body(i, c):
        tab_v[pl.ds(i * _LANES, _LANES)] = jnp.zeros((_LANES,), jnp.int32)
        return c
    lax.fori_loop(0, 256 // _LANES, zero_body, 0)

    ones = jnp.ones((_LANES,), jnp.int32)
    mask = jnp.full((_LANES,), 255, jnp.int32)

    def body(j, c):
        r = j // 4
        c0 = (j % 4) * (_W // 4)
        ws = [ids_v[r, pl.ds(c0 + cc * _LANES, _LANES)]
              for cc in range(_W // _LANES // 4)]
        for w in ws:
            for k in range(4):
                bk = lax.shift_right_logical(w, 8 * k) & mask
                plsc.addupdate_scatter(tab_v, [bk], ones)
        return c
    lax.fori_loop(0, _QROWS * 4, body, 0)

    pltpu.sync_copy(tab_v, hist_hbm.at[quarter * _HB + batch])


def _sc_hist(ids_packed):
    kern = functools.partial(
        pl.kernel,
        out_type=jax.ShapeDtypeStruct((4 * _HB, 256), jnp.int32),
        mesh=plsc.VectorSubcoreMesh(core_axis_name="c", subcore_axis_name="s"),
        compiler_params=pltpu.CompilerParams(needs_layout_passes=False),
        scratch_types=[
            pltpu.VMEM((_QROWS, _W), jnp.int32),
            pltpu.VMEM((256,), jnp.int32),
        ],
    )(_sc_hist_kernel)
    return kern(ids_packed)


def _combine_kernel(ha_ref, hb_ref, lsa_ref, lsb_ref, dstats_ref,
                    wr_ref, wd_ref):
    ha = ha_ref[...].astype(jnp.float32)
    hb = hb_ref[...].astype(jnp.float32)
    ha8 = ha[0:8] + ha[8:16] + ha[16:24] + ha[24:32]
    hb8 = hb[0:8] + hb[8:16] + hb[16:24] + hb[24:32]
    h16 = jnp.concatenate([ha8, hb8], axis=0)
    p = h16 * (1.0 / _N)
    ent = -jnp.sum(p * (jnp.log(p + _EPS) * (1.0 / 0.6931471805599453)),
                   axis=1, keepdims=True)

    s = jnp.concatenate([jnp.squeeze(lsa_ref[...], 1),
                         jnp.squeeze(lsb_ref[...], 1)], axis=0)
    t = jnp.squeeze(dstats_ref[...], 1)
    lsum = s[:, 0:1]
    lsq = s[:, 1:2]
    n = t[:, 0:1]
    s1 = t[:, 1:2]
    s2 = t[:, 2:3]

    lapvar = (lsq - lsum * lsum * (1.0 / _N)) / (_N - 1.0)
    clarity = lapvar / (_MAX_LAPLACE_VAR + _EPS)
    uniformity = 1.0 / (ent + _EPS)
    rgb_score = 0.5 * (clarity + uniformity)

    mean = s1 / jnp.maximum(n, 1.0)
    var = (s2 - 2.0 * mean * s1 + mean * mean * n) / jnp.maximum(n - 1.0, 1.0)
    noise = jnp.where(n > 0.0, jnp.sqrt(var), 1.0)
    density = n * (1.0 / _N)
    depth_score = 0.5 * (density / (_MAX_DENSITY + _EPS) + 1.0 / (noise + _EPS))

    den = rgb_score + depth_score + _EPS
    wr_ref[...] = rgb_score / den
    wd_ref[...] = depth_score / den


def _combine(hist_a, hist_b, lstats_a, lstats_b, dstats):
    return pl.pallas_call(
        _combine_kernel,
        out_shape=[
            jax.ShapeDtypeStruct((16, 1), jnp.float32),
            jax.ShapeDtypeStruct((16, 1), jnp.float32),
        ],
    )(hist_a, hist_b, lstats_a, lstats_b, dstats)


def kernel(rgb_img, depth_img):
    ids_a, lstats_a = _tc_rgb(rgb_img, 0)
    hist_a = _sc_hist(ids_a)
    ids_b, lstats_b = _tc_rgb(rgb_img, _HB // 2)
    hist_b = _sc_hist(ids_b)
    dstats, = _tc_depth(depth_img)
    wr, wd = _combine(hist_a, hist_b, lstats_a, lstats_b, dstats)
    return wr, wd


# speedup vs baseline: 1.0370x; 1.0218x over previous
import functools

import jax
import jax.numpy as jnp
from jax import lax
from jax.experimental import pallas as pl
from jax.experimental.pallas import tpu as pltpu
from jax.experimental.pallas import tpu_sc as plsc

_EPS = 1e-4
_MAX_LAPLACE_VAR = 1000.0
_MAX_DENSITY = 10000.0

_H = 512
_W = 512
_N = float(_H * _W)
_LANES = 16
_HB = 8
_QROWS = (_H // 4) // 4


def _rgb_kernel(rgb_ref, ids_ref, stats_ref):
    r = rgb_ref[:, 0]
    g = rgb_ref[:, 1]
    b = rgb_ref[:, 2]
    gray = 0.299 * r + 0.587 * g + 0.114 * b

    zrow = jnp.zeros((2, 1, _W), jnp.float32)
    zcol = jnp.zeros((2, _H, 1), jnp.float32)
    dn = jnp.concatenate([gray[:, 1:], zrow], axis=1)
    up = jnp.concatenate([zrow, gray[:, :-1]], axis=1)
    rt = jnp.concatenate([gray[:, :, 1:], zcol], axis=2)
    lt = jnp.concatenate([zcol, gray[:, :, :-1]], axis=2)
    lap = dn + up + rt + lt - 4.0 * gray
    lsum = jnp.sum(jnp.sum(lap, axis=2), axis=1)
    lsq = jnp.sum(jnp.sum(lap * lap, axis=2), axis=1)

    ids = jnp.clip(gray * 255.0, 0.0, 255.0).astype(jnp.int32)
    idr = ids.reshape(2, 4, _H // 4, _W)
    packed = (idr[:, 0] | (idr[:, 1] << 8)
              | (idr[:, 2] << 16) | (idr[:, 3] << 24))
    ids_ref[...] = packed

    lane = lax.broadcasted_iota(jnp.int32, (2, 1, 128), 2)
    vals = (jnp.where(lane == 0, lsum.reshape(2, 1, 1), 0.0)
            + jnp.where(lane == 1, lsq.reshape(2, 1, 1), 0.0))
    stats_ref[...] = vals


def _tc_rgb(rgb_img, off):
    return pl.pallas_call(
        _rgb_kernel,
        grid=(_HB // 2,),
        in_specs=[
            pl.BlockSpec((2, 3, _H, _W), lambda i: (i + off, 0, 0, 0)),
        ],
        out_specs=[
            pl.BlockSpec((2, _H // 4, _W), lambda i: (i, 0, 0)),
            pl.BlockSpec((2, 1, 128), lambda i: (i, 0, 0)),
        ],
        out_shape=[
            jax.ShapeDtypeStruct((_HB, _H // 4, _W), jnp.int32),
            jax.ShapeDtypeStruct((_HB, 1, 128), jnp.float32),
        ],
    )(rgb_img)


def _depth_kernel(depth_ref, stats_ref):
    d = depth_ref[:, 0]
    m = d > 0.0
    n = jnp.sum(jnp.sum(m.astype(jnp.float32), axis=2), axis=1)
    s1 = jnp.sum(jnp.sum(jnp.where(m, d, 0.0), axis=2), axis=1)
    s2 = jnp.sum(jnp.sum(jnp.where(m, d * d, 0.0), axis=2), axis=1)

    lane = lax.broadcasted_iota(jnp.int32, (4, 1, 128), 2)
    vals = (jnp.where(lane == 0, n.reshape(4, 1, 1), 0.0)
            + jnp.where(lane == 1, s1.reshape(4, 1, 1), 0.0)
            + jnp.where(lane == 2, s2.reshape(4, 1, 1), 0.0))
    stats_ref[...] = vals


def _tc_depth(depth_img):
    B = depth_img.shape[0]
    return pl.pallas_call(
        _depth_kernel,
        grid=(B // 4,),
        in_specs=[
            pl.BlockSpec((4, 1, _H, _W), lambda i: (i, 0, 0, 0)),
        ],
        out_specs=[
            pl.BlockSpec((4, 1, 128), lambda i: (i, 0, 0)),
        ],
        out_shape=[
            jax.ShapeDtypeStruct((B, 1, 128), jnp.float32),
        ],
    )(depth_img)


def _sc_hist_kernel(ids_hbm, hist_hbm, ids_v, tab_v):
    wid = lax.axis_index("s") * 2 + lax.axis_index("c")
    batch = wid % _HB
    quarter = wid // _HB

    pltpu.sync_copy(ids_hbm.at[batch, pl.ds(quarter * _QROWS, _QROWS)], ids_v)

    def zero_body(i, c):
        tab_v[pl.ds(i * _LANES, _LANES)] = jnp.zeros((_LANES,), jnp.int32)
        return c
    lax.fori_loop(0, 256 // _LANES, zero_body, 0)

    ones = jnp.ones((_LANES,), jnp.int32)
    mask = jnp.full((_LANES,), 255, jnp.int32)

    def body(j, c):
        r = j // 4
        c0 = (j % 4) * (_W // 4)
        ws = [ids_v[r, pl.ds(c0 + cc * _LANES, _LANES)]
              for cc in range(_W // _LANES // 4)]
        for w in ws:
            for k in range(4):
                bk = lax.shift_right_logical(w, 8 * k) & mask
                plsc.addupdate_scatter(tab_v, [bk], ones)
        return c
    lax.fori_loop(0, _QROWS * 4, body, 0)

    pltpu.sync_copy(tab_v, hist_hbm.at[quarter * _HB + batch])


def _sc_hist(ids_packed):
    kern = functools.partial(
        pl.kernel,
        out_type=jax.ShapeDtypeStruct((4 * _HB, 256), jnp.int32),
        mesh=plsc.VectorSubcoreMesh(core_axis_name="c", subcore_axis_name="s"),
        compiler_params=pltpu.CompilerParams(needs_layout_passes=False),
        scratch_types=[
            pltpu.VMEM((_QROWS, _W), jnp.int32),
            pltpu.VMEM((256,), jnp.int32),
        ],
    )(_sc_hist_kernel)
    return kern(ids_packed)


def _combine_kernel(ha_ref, hb_ref, lsa_ref, lsb_ref, dstats_ref, w_ref):
    ha = ha_ref[...].astype(jnp.float32)
    hb = hb_ref[...].astype(jnp.float32)
    ha8 = ha[0:8] + ha[8:16] + ha[16:24] + ha[24:32]
    hb8 = hb[0:8] + hb[8:16] + hb[16:24] + hb[24:32]
    h16 = jnp.concatenate([ha8, hb8], axis=0)
    p = h16 * (1.0 / _N)
    ent = -jnp.sum(p * (jnp.log(p + _EPS) * (1.0 / 0.6931471805599453)),
                   axis=1, keepdims=True)

    s = jnp.concatenate([jnp.squeeze(lsa_ref[...], 1),
                         jnp.squeeze(lsb_ref[...], 1)], axis=0)
    t = jnp.squeeze(dstats_ref[...], 1)
    lsum = s[:, 0:1]
    lsq = s[:, 1:2]
    n = t[:, 0:1]
    s1 = t[:, 1:2]
    s2 = t[:, 2:3]

    lapvar = (lsq - lsum * lsum * (1.0 / _N)) / (_N - 1.0)
    clarity = lapvar / (_MAX_LAPLACE_VAR + _EPS)
    uniformity = 1.0 / (ent + _EPS)
    rgb_score = 0.5 * (clarity + uniformity)

    mean = s1 / jnp.maximum(n, 1.0)
    var = (s2 - 2.0 * mean * s1 + mean * mean * n) / jnp.maximum(n - 1.0, 1.0)
    noise = jnp.where(n > 0.0, jnp.sqrt(var), 1.0)
    density = n * (1.0 / _N)
    depth_score = 0.5 * (density / (_MAX_DENSITY + _EPS) + 1.0 / (noise + _EPS))

    den = rgb_score + depth_score + _EPS
    w_ref[...] = jnp.concatenate([(rgb_score / den).reshape(1, 16, 1),
                                  (depth_score / den).reshape(1, 16, 1)],
                                 axis=0)


def _combine(hist_a, hist_b, lstats_a, lstats_b, dstats):
    return pl.pallas_call(
        _combine_kernel,
        out_shape=jax.ShapeDtypeStruct((2, 16, 1), jnp.float32),
    )(hist_a, hist_b, lstats_a, lstats_b, dstats)


def kernel(rgb_img, depth_img):
    ids_a, lstats_a = _tc_rgb(rgb_img, 0)
    hist_a = _sc_hist(ids_a)
    ids_b, lstats_b = _tc_rgb(rgb_img, _HB // 2)
    hist_b = _sc_hist(ids_b)
    dstats, = _tc_depth(depth_img)
    w = _combine(hist_a, hist_b, lstats_a, lstats_b, dstats)
    return w[0], w[1]


# speedup vs baseline: 1.0642x; 1.0262x over previous
import functools

import jax
import jax.numpy as jnp
from jax import lax
from jax.experimental import pallas as pl
from jax.experimental.pallas import tpu as pltpu
from jax.experimental.pallas import tpu_sc as plsc

_EPS = 1e-4
_MAX_LAPLACE_VAR = 1000.0
_MAX_DENSITY = 10000.0

_H = 512
_W = 512
_N = float(_H * _W)
_LANES = 16
_HB = 16
_QROWS = (_H // 4) // 2


def _rgb_kernel(rgb_ref, ids_ref, stats_ref):
    r = rgb_ref[:, 0]
    g = rgb_ref[:, 1]
    b = rgb_ref[:, 2]
    gray = 0.299 * r + 0.587 * g + 0.114 * b

    zrow = jnp.zeros((2, 1, _W), jnp.float32)
    zcol = jnp.zeros((2, _H, 1), jnp.float32)
    dn = jnp.concatenate([gray[:, 1:], zrow], axis=1)
    up = jnp.concatenate([zrow, gray[:, :-1]], axis=1)
    rt = jnp.concatenate([gray[:, :, 1:], zcol], axis=2)
    lt = jnp.concatenate([zcol, gray[:, :, :-1]], axis=2)
    lap = dn + up + rt + lt - 4.0 * gray
    lsum = jnp.sum(jnp.sum(lap, axis=2), axis=1)
    lsq = jnp.sum(jnp.sum(lap * lap, axis=2), axis=1)

    ids = jnp.clip(gray * 255.0, 0.0, 255.0).astype(jnp.int32)
    idr = ids.reshape(2, 4, _H // 4, _W)
    packed = (idr[:, 0] | (idr[:, 1] << 8)
              | (idr[:, 2] << 16) | (idr[:, 3] << 24))
    ids_ref[...] = packed

    lane = lax.broadcasted_iota(jnp.int32, (2, 1, 128), 2)
    vals = (jnp.where(lane == 0, lsum.reshape(2, 1, 1), 0.0)
            + jnp.where(lane == 1, lsq.reshape(2, 1, 1), 0.0))
    stats_ref[...] = vals


def _tc_rgb(rgb_img, off):
    return pl.pallas_call(
        _rgb_kernel,
        grid=(_HB // 2,),
        in_specs=[
            pl.BlockSpec((2, 3, _H, _W), lambda i: (i + off, 0, 0, 0)),
        ],
        out_specs=[
            pl.BlockSpec((2, _H // 4, _W), lambda i: (i, 0, 0)),
            pl.BlockSpec((2, 1, 128), lambda i: (i, 0, 0)),
        ],
        out_shape=[
            jax.ShapeDtypeStruct((_HB, _H // 4, _W), jnp.int32),
            jax.ShapeDtypeStruct((_HB, 1, 128), jnp.float32),
        ],
    )(rgb_img)


def _depth_kernel(depth_ref, stats_ref):
    d = depth_ref[:, 0]
    m = d > 0.0
    n = jnp.sum(jnp.sum(m.astype(jnp.float32), axis=2), axis=1)
    s1 = jnp.sum(jnp.sum(jnp.where(m, d, 0.0), axis=2), axis=1)
    s2 = jnp.sum(jnp.sum(jnp.where(m, d * d, 0.0), axis=2), axis=1)

    lane = lax.broadcasted_iota(jnp.int32, (4, 1, 128), 2)
    vals = (jnp.where(lane == 0, n.reshape(4, 1, 1), 0.0)
            + jnp.where(lane == 1, s1.reshape(4, 1, 1), 0.0)
            + jnp.where(lane == 2, s2.reshape(4, 1, 1), 0.0))
    stats_ref[...] = vals


def _tc_depth(depth_img):
    B = depth_img.shape[0]
    return pl.pallas_call(
        _depth_kernel,
        grid=(B // 4,),
        in_specs=[
            pl.BlockSpec((4, 1, _H, _W), lambda i: (i, 0, 0, 0)),
        ],
        out_specs=[
            pl.BlockSpec((4, 1, 128), lambda i: (i, 0, 0)),
        ],
        out_shape=[
            jax.ShapeDtypeStruct((B, 1, 128), jnp.float32),
        ],
    )(depth_img)


def _sc_hist_kernel(ids_hbm, hist_hbm, ids_v, tab_v):
    wid = lax.axis_index("s") * 2 + lax.axis_index("c")
    batch = wid % _HB
    quarter = wid // _HB

    pltpu.sync_copy(ids_hbm.at[batch, pl.ds(quarter * _QROWS, _QROWS)], ids_v)

    def zero_body(i, c):
        tab_v[pl.ds(i * _LANES, _LANES)] = jnp.zeros((_LANES,), jnp.int32)
        return c
    lax.fori_loop(0, 256 // _LANES, zero_body, 0)

    ones = jnp.ones((_LANES,), jnp.int32)
    mask = jnp.full((_LANES,), 255, jnp.int32)

    def body(j, c):
        r = j // 4
        c0 = (j % 4) * (_W // 4)
        ws = [ids_v[r, pl.ds(c0 + cc * _LANES, _LANES)]
              for cc in range(_W // _LANES // 4)]
        for w in ws:
            for k in range(4):
                bk = lax.shift_right_logical(w, 8 * k) & mask
                plsc.addupdate_scatter(tab_v, [bk], ones)
        return c
    lax.fori_loop(0, _QROWS * 4, body, 0)

    pltpu.sync_copy(tab_v, hist_hbm.at[quarter * _HB + batch])


def _sc_hist(ids_packed):
    kern = functools.partial(
        pl.kernel,
        out_type=jax.ShapeDtypeStruct((2 * _HB, 256), jnp.int32),
        mesh=plsc.VectorSubcoreMesh(core_axis_name="c", subcore_axis_name="s"),
        compiler_params=pltpu.CompilerParams(needs_layout_passes=False),
        scratch_types=[
            pltpu.VMEM((_QROWS, _W), jnp.int32),
            pltpu.VMEM((256,), jnp.int32),
        ],
    )(_sc_hist_kernel)
    return kern(ids_packed)


def _combine_kernel(h_ref, ls_ref, dstats_ref, w_ref):
    h = h_ref[...].astype(jnp.float32)
    h16 = h[0:16] + h[16:32]
    p = h16 * (1.0 / _N)
    ent = -jnp.sum(p * (jnp.log(p + _EPS) * (1.0 / 0.6931471805599453)),
                   axis=1, keepdims=True)

    s = jnp.squeeze(ls_ref[...], 1)
    t = jnp.squeeze(dstats_ref[...], 1)
    lsum = s[:, 0:1]
    lsq = s[:, 1:2]
    n = t[:, 0:1]
    s1 = t[:, 1:2]
    s2 = t[:, 2:3]

    lapvar = (lsq - lsum * lsum * (1.0 / _N)) / (_N - 1.0)
    clarity = lapvar / (_MAX_LAPLACE_VAR + _EPS)
    uniformity = 1.0 / (ent + _EPS)
    rgb_score = 0.5 * (clarity + uniformity)

    mean = s1 / jnp.maximum(n, 1.0)
    var = (s2 - 2.0 * mean * s1 + mean * mean * n) / jnp.maximum(n - 1.0, 1.0)
    noise = jnp.where(n > 0.0, jnp.sqrt(var), 1.0)
    density = n * (1.0 / _N)
    depth_score = 0.5 * (density / (_MAX_DENSITY + _EPS) + 1.0 / (noise + _EPS))

    den = rgb_score + depth_score + _EPS
    w_ref[...] = jnp.concatenate([(rgb_score / den).reshape(1, 16, 1),
                                  (depth_score / den).reshape(1, 16, 1)],
                                 axis=0)


def _combine(hist, lstats, dstats):
    return pl.pallas_call(
        _combine_kernel,
        out_shape=jax.ShapeDtypeStruct((2, 16, 1), jnp.float32),
    )(hist, lstats, dstats)


def kernel(rgb_img, depth_img):
    ids, lstats = _tc_rgb(rgb_img, 0)
    hist = _sc_hist(ids)
    dstats, = _tc_depth(depth_img)
    w = _combine(hist, lstats, dstats)
    return w[0], w[1]


# speedup vs baseline: 1.0800x; 1.0149x over previous
import functools

import jax
import jax.numpy as jnp
from jax import lax
from jax.experimental import pallas as pl
from jax.experimental.pallas import tpu as pltpu
from jax.experimental.pallas import tpu_sc as plsc

_EPS = 1e-4
_MAX_LAPLACE_VAR = 1000.0
_MAX_DENSITY = 10000.0

_H = 512
_W = 512
_N = float(_H * _W)
_LANES = 16
_HB = 16
_QROWS = (_H // 4) // 2


def _rgb_kernel(rgb_ref, ids_ref, stats_ref):
    r = rgb_ref[:, 0]
    g = rgb_ref[:, 1]
    b = rgb_ref[:, 2]
    gray = 0.299 * r + 0.587 * g + 0.114 * b

    zrow = jnp.zeros((4, 1, _W), jnp.float32)
    zcol = jnp.zeros((4, _H, 1), jnp.float32)
    dn = jnp.concatenate([gray[:, 1:], zrow], axis=1)
    up = jnp.concatenate([zrow, gray[:, :-1]], axis=1)
    rt = jnp.concatenate([gray[:, :, 1:], zcol], axis=2)
    lt = jnp.concatenate([zcol, gray[:, :, :-1]], axis=2)
    lap = dn + up + rt + lt - 4.0 * gray
    lsum = jnp.sum(jnp.sum(lap, axis=2), axis=1)
    lsq = jnp.sum(jnp.sum(lap * lap, axis=2), axis=1)

    ids = jnp.clip(gray * 255.0, 0.0, 255.0).astype(jnp.int32)
    idr = ids.reshape(4, 4, _H // 4, _W)
    packed = (idr[:, 0] | (idr[:, 1] << 8)
              | (idr[:, 2] << 16) | (idr[:, 3] << 24))
    ids_ref[...] = packed

    lane = lax.broadcasted_iota(jnp.int32, (4, 1, 128), 2)
    vals = (jnp.where(lane == 0, lsum.reshape(4, 1, 1), 0.0)
            + jnp.where(lane == 1, lsq.reshape(4, 1, 1), 0.0))
    stats_ref[...] = vals


def _tc_rgb(rgb_img, off):
    return pl.pallas_call(
        _rgb_kernel,
        grid=(_HB // 4,),
        in_specs=[
            pl.BlockSpec((4, 3, _H, _W), lambda i: (i + off, 0, 0, 0)),
        ],
        out_specs=[
            pl.BlockSpec((4, _H // 4, _W), lambda i: (i, 0, 0)),
            pl.BlockSpec((4, 1, 128), lambda i: (i, 0, 0)),
        ],
        out_shape=[
            jax.ShapeDtypeStruct((_HB, _H // 4, _W), jnp.int32),
            jax.ShapeDtypeStruct((_HB, 1, 128), jnp.float32),
        ],
    )(rgb_img)


def _depth_kernel(depth_ref, stats_ref):
    d = depth_ref[:, 0]
    m = d > 0.0
    n = jnp.sum(jnp.sum(m.astype(jnp.float32), axis=2), axis=1)
    s1 = jnp.sum(jnp.sum(jnp.where(m, d, 0.0), axis=2), axis=1)
    s2 = jnp.sum(jnp.sum(jnp.where(m, d * d, 0.0), axis=2), axis=1)

    lane = lax.broadcasted_iota(jnp.int32, (4, 1, 128), 2)
    vals = (jnp.where(lane == 0, n.reshape(4, 1, 1), 0.0)
            + jnp.where(lane == 1, s1.reshape(4, 1, 1), 0.0)
            + jnp.where(lane == 2, s2.reshape(4, 1, 1), 0.0))
    stats_ref[...] = vals


def _tc_depth(depth_img):
    B = depth_img.shape[0]
    return pl.pallas_call(
        _depth_kernel,
        grid=(B // 4,),
        in_specs=[
            pl.BlockSpec((4, 1, _H, _W), lambda i: (i, 0, 0, 0)),
        ],
        out_specs=[
            pl.BlockSpec((4, 1, 128), lambda i: (i, 0, 0)),
        ],
        out_shape=[
            jax.ShapeDtypeStruct((B, 1, 128), jnp.float32),
        ],
    )(depth_img)


def _sc_hist_kernel(ids_hbm, hist_hbm, ids_v, tab_v):
    wid = lax.axis_index("s") * 2 + lax.axis_index("c")
    batch = wid % _HB
    quarter = wid // _HB

    pltpu.sync_copy(ids_hbm.at[batch, pl.ds(quarter * _QROWS, _QROWS)], ids_v)

    def zero_body(i, c):
        tab_v[pl.ds(i * _LANES, _LANES)] = jnp.zeros((_LANES,), jnp.int32)
        return c
    lax.fori_loop(0, 256 // _LANES, zero_body, 0)

    ones = jnp.ones((_LANES,), jnp.int32)
    mask = jnp.full((_LANES,), 255, jnp.int32)

    def body(j, c):
        r = j // 4
        c0 = (j % 4) * (_W // 4)
        ws = [ids_v[r, pl.ds(c0 + cc * _LANES, _LANES)]
              for cc in range(_W // _LANES // 4)]
        for w in ws:
            for k in range(4):
                bk = lax.shift_right_logical(w, 8 * k) & mask
                plsc.addupdate_scatter(tab_v, [bk], ones)
        return c
    lax.fori_loop(0, _QROWS * 4, body, 0)

    pltpu.sync_copy(tab_v, hist_hbm.at[quarter * _HB + batch])


def _sc_hist(ids_packed):
    kern = functools.partial(
        pl.kernel,
        out_type=jax.ShapeDtypeStruct((2 * _HB, 256), jnp.int32),
        mesh=plsc.VectorSubcoreMesh(core_axis_name="c", subcore_axis_name="s"),
        compiler_params=pltpu.CompilerParams(needs_layout_passes=False),
        scratch_types=[
            pltpu.VMEM((_QROWS, _W), jnp.int32),
            pltpu.VMEM((256,), jnp.int32),
        ],
    )(_sc_hist_kernel)
    return kern(ids_packed)


def _combine_kernel(h_ref, ls_ref, dstats_ref, w_ref):
    h = h_ref[...].astype(jnp.float32)
    h16 = h[0:16] + h[16:32]
    p = h16 * (1.0 / _N)
    ent = -jnp.sum(p * (jnp.log(p + _EPS) * (1.0 / 0.6931471805599453)),
                   axis=1, keepdims=True)

    s = jnp.squeeze(ls_ref[...], 1)
    t = jnp.squeeze(dstats_ref[...], 1)
    lsum = s[:, 0:1]
    lsq = s[:, 1:2]
    n = t[:, 0:1]
    s1 = t[:, 1:2]
    s2 = t[:, 2:3]

    lapvar = (lsq - lsum * lsum * (1.0 / _N)) / (_N - 1.0)
    clarity = lapvar / (_MAX_LAPLACE_VAR + _EPS)
    uniformity = 1.0 / (ent + _EPS)
    rgb_score = 0.5 * (clarity + uniformity)

    mean = s1 / jnp.maximum(n, 1.0)
    var = (s2 - 2.0 * mean * s1 + mean * mean * n) / jnp.maximum(n - 1.0, 1.0)
    noise = jnp.where(n > 0.0, jnp.sqrt(var), 1.0)
    density = n * (1.0 / _N)
    depth_score = 0.5 * (density / (_MAX_DENSITY + _EPS) + 1.0 / (noise + _EPS))

    den = rgb_score + depth_score + _EPS
    w_ref[...] = jnp.concatenate([(rgb_score / den).reshape(1, 16, 1),
                                  (depth_score / den).reshape(1, 16, 1)],
                                 axis=0)


def _combine(hist, lstats, dstats):
    return pl.pallas_call(
        _combine_kernel,
        out_shape=jax.ShapeDtypeStruct((2, 16, 1), jnp.float32),
    )(hist, lstats, dstats)


def kernel(rgb_img, depth_img):
    ids, lstats = _tc_rgb(rgb_img, 0)
    hist = _sc_hist(ids)
    dstats, = _tc_depth(depth_img)
    w = _combine(hist, lstats, dstats)
    return w[0], w[1]
